# baseline (speedup 1.0000x reference)
import functools

import jax
import jax.numpy as jnp
from jax import lax
from jax.experimental import pallas as pl
from jax.experimental.pallas import tpu as pltpu
from jax.experimental.pallas import tpu_sc as plsc

N = 10000
E = 320000
D = 128
NSUB = 16
NP = N + 112
ROWS_PER_SUB = NP // NSUB
CHUNK = 128
NBUF = 2
GRP = 8
NCHUNK = 160
EDGES_PER_SUB = CHUNK * NCHUNK
E_PAD = EDGES_PER_SUB * NSUB


def _sc_accumulate(lbls, src, dst, zeros):
    mesh = plsc.VectorSubcoreMesh(core_axis_name="c", subcore_axis_name="s")

    @functools.partial(
        pl.kernel,
        out_type=(
            jax.ShapeDtypeStruct((2, NP, D), jnp.float32),
            jax.ShapeDtypeStruct((2 * NP,), jnp.float32),
        ),
        mesh=mesh,
        scratch_types=[
            pltpu.VMEM_SHARED((NP, D), jnp.float32),
            pltpu.VMEM_SHARED((NP,), jnp.float32),
            pltpu.VMEM((CHUNK,), jnp.float32),
            pltpu.VMEM((640,), jnp.float32),
            pltpu.VMEM((GRP, CHUNK), jnp.int32),
            pltpu.VMEM((GRP, CHUNK), jnp.int32),
            [pltpu.VMEM((CHUNK, D), jnp.float32) for _ in range(NBUF)],
            [pltpu.SemaphoreType.DMA for _ in range(NBUF)],
            [pltpu.SemaphoreType.DMA for _ in range(NBUF)],
            [pltpu.SemaphoreType.DMA for _ in range(NBUF)],
            pltpu.SemaphoreType.DMA,
        ],
    )
    def body(lbl_hbm, src_hbm, dst_hbm, z_hbm, sum_hbm, cnt_hbm,
             acc, cnt_sh, ones, stage, sidx, didx, rows, gsems, ssems,
             csems, isem):
        c = lax.axis_index("c")
        s = lax.axis_index("s")
        r0 = s * ROWS_PER_SUB
        row0 = (c * NSUB + s) * NCHUNK
        pltpu.sync_copy(z_hbm, acc.at[pl.ds(r0, ROWS_PER_SUB)])
        ones16 = jnp.full((16,), 1.0, jnp.float32)
        for k in range(CHUNK // 16):
            ones[pl.ds(k * 16, 16)] = ones16
        zero16 = jnp.zeros((16,), jnp.float32)
        for k in range(640 // 16):
            stage[pl.ds(k * 16, 16)] = zero16
        pltpu.sync_copy(stage.at[pl.ds(0, ROWS_PER_SUB)],
                        cnt_sh.at[pl.ds(r0, ROWS_PER_SUB)])
        plsc.subcore_barrier()

        def step(t, carry):
            i1 = pltpu.async_copy(
                src_hbm.at[pl.ds(row0 + t * GRP, GRP)], sidx, isem)
            i2 = pltpu.async_copy(
                dst_hbm.at[pl.ds(row0 + t * GRP, GRP)], didx, isem)
            i1.wait()
            i2.wait()
            gets = {}
            for b in range(NBUF):
                gets[b] = pltpu.async_copy(
                    lbl_hbm.at[sidx.at[b]], rows[b], gsems[b])
            for k in range(GRP):
                b = k % NBUF
                gets[b].wait()
                put = pltpu.async_copy(
                    rows[b], acc.at[didx.at[k]], ssems[b], add=True)
                cnt = pltpu.async_copy(
                    ones, cnt_sh.at[didx.at[k]], csems[b], add=True)
                put.wait()
                cnt.wait()
                if k + NBUF < GRP:
                    gets[b] = pltpu.async_copy(
                        lbl_hbm.at[sidx.at[k + NBUF]], rows[b], gsems[b])
            return carry

        lax.fori_loop(0, NCHUNK // GRP, step, 0)
        plsc.subcore_barrier()
        pltpu.sync_copy(acc.at[pl.ds(r0, ROWS_PER_SUB)],
                        sum_hbm.at[c, pl.ds(r0, ROWS_PER_SUB)])
        pltpu.sync_copy(cnt_sh.at[pl.ds(r0, ROWS_PER_SUB)],
                        stage.at[pl.ds(0, ROWS_PER_SUB)])
        pltpu.sync_copy(stage.at[pl.ds(0, ROWS_PER_SUB)],
                        cnt_hbm.at[pl.ds(c * NP + r0, ROWS_PER_SUB)])

    return body(lbls, src, dst, zeros)


def _tc_combine(sums_ref, cnts_ref, lbls_ref, msk_ref, out_ref):
    c1 = jnp.maximum(cnts_ref[0][:N, :], 1.0)
    c2 = jnp.maximum(cnts_ref[1][:N, :], 1.0)
    m = 0.5 * (sums_ref[0][:N, :] / c1 + sums_ref[1][:N, :] / c2)
    out_ref[...] = jnp.where(msk_ref[...] > 0, m, lbls_ref[...])


def kernel(lbls, no_lbl_idx, knn_sc, knn_fc):
    pad = E_PAD - E
    pad_dst = (jnp.arange(pad, dtype=jnp.int32) % (NP - N)) + N
    zpad = jnp.zeros((pad,), jnp.int32)
    src = jnp.concatenate(
        [knn_sc[0], zpad, knn_fc[0], zpad]).reshape(2 * NSUB * NCHUNK, CHUNK)
    dst = jnp.concatenate(
        [knn_sc[1], pad_dst, knn_fc[1], pad_dst]).reshape(
            2 * NSUB * NCHUNK, CHUNK)
    zeros = jnp.zeros((ROWS_PER_SUB, D), jnp.float32)

    sums, cnts = _sc_accumulate(lbls, src, dst, zeros)

    msk = no_lbl_idx.astype(jnp.int32).reshape(N, 1)
    return pl.pallas_call(
        _tc_combine,
        out_shape=jax.ShapeDtypeStruct((N, D), jnp.float32),
    )(sums, cnts.reshape(2, NP, 1), lbls, msk)
